# baseline (speedup 1.0000x reference)
import functools

import jax
import jax.numpy as jnp
from jax import lax
from jax.experimental import pallas as pl
from jax.experimental.pallas import tpu as pltpu
from jax.experimental.pallas import tpu_sc as plsc

_N = 8192
_D = 128
_B = 4096
_NEG = 5
_NW = 32
_BPW = _B // _NW
_L = 16
_NBLK = _BPW // _L


@functools.partial(
    pl.kernel,
    mesh=plsc.VectorSubcoreMesh(core_axis_name="c", subcore_axis_name="s"),
    compiler_params=pltpu.CompilerParams(needs_layout_passes=False),
    out_type=jax.ShapeDtypeStruct((_NW, 6, _L), jnp.float32),
    scratch_types=[
        pltpu.VMEM((_BPW,), jnp.int32),
        pltpu.VMEM((_BPW,), jnp.int32),
        pltpu.VMEM((_NEG, _BPW), jnp.int32),
        pltpu.VMEM((_BPW,), jnp.int32),
        pltpu.VMEM((_NEG, _BPW), jnp.int32),
        pltpu.VMEM((_NEG, _BPW), jnp.int32),
        pltpu.VMEM((_BPW, _D), jnp.float32),
        pltpu.VMEM((_BPW, _D), jnp.float32),
        pltpu.VMEM((_NEG, _BPW, _D), jnp.float32),
        pltpu.VMEM((_BPW,), jnp.float32),
        pltpu.VMEM((_NEG, _BPW), jnp.float32),
        pltpu.VMEM((_NEG, _BPW), jnp.float32),
        pltpu.VMEM((6, _L), jnp.float32),
        pltpu.SemaphoreType.DMA,
    ],
)
def _nsloss_sc(idx_s_hbm, idx_t_hbm, idx_neg_hbm, emb_s_hbm, emb_t_hbm,
               adjm_hbm, adjt_hbm, out_hbm,
               idxs_v, idxt_v, idxn_v, linp_v, linst_v, lintt_v,
               xs_v, xt_v, xn_v, yp_v, yst_v, ytt_v, acc_v, sem):
    wid = lax.axis_index("s") * 2 + lax.axis_index("c")
    base = wid * _BPW

    pltpu.sync_copy(idx_s_hbm.at[pl.ds(base, _BPW)], idxs_v)
    pltpu.sync_copy(idx_t_hbm.at[pl.ds(base, _BPW)], idxt_v)
    for n in range(_NEG):
        pltpu.sync_copy(idx_neg_hbm.at[pl.ds(n * _B + base, _BPW)], idxn_v.at[n])

    cps = [
        pltpu.async_copy(emb_s_hbm.at[idxs_v], xs_v, sem),
        pltpu.async_copy(emb_t_hbm.at[idxt_v], xt_v, sem),
    ]
    for n in range(_NEG):
        cps.append(pltpu.async_copy(emb_t_hbm.at[idxn_v.at[n]], xn_v.at[n], sem))

    for c in range(_NBLK):
        sl = pl.ds(c * _L, _L)
        s16 = idxs_v[sl]
        t16 = idxt_v[sl]
        linp_v[sl] = s16 * _N + t16
        for n in range(_NEG):
            n16 = idxn_v[n, sl]
            linst_v[n, sl] = s16 * _N + n16
            lintt_v[n, sl] = t16 * _N + n16

    cps.append(pltpu.async_copy(adjm_hbm.at[linp_v], yp_v, sem))
    for n in range(_NEG):
        cps.append(pltpu.async_copy(adjm_hbm.at[linst_v.at[n]], yst_v.at[n], sem))
        cps.append(pltpu.async_copy(adjt_hbm.at[lintt_v.at[n]], ytt_v.at[n], sem))
    for cp in cps:
        cp.wait()

    iota = lax.iota(jnp.int32, _L)
    zero = jnp.zeros((_L,), jnp.float32)
    accs = [zero] * 6

    for blk in range(_NBLK):
        b0 = blk * _L
        ii = b0 + iota

        def kbody(k, c, ii=ii):
            kk = jnp.full((_L,), k, jnp.int32)
            vs = plsc.load_gather(xs_v, [ii, kk])
            vt = plsc.load_gather(xt_v, [ii, kk])
            out = [c[0] + vs * vt]
            for n in range(_NEG):
                nn = jnp.full((_L,), n, jnp.int32)
                vn = plsc.load_gather(xn_v, [nn, ii, kk])
                out.append(c[1 + 2 * n] + vs * vn)
                out.append(c[2 + 2 * n] + vt * vn)
            return tuple(out)

        init = (zero,) * (1 + 2 * _NEG)
        dots = lax.fori_loop(0, _D, kbody, init)

        dp = dots[0]
        yp = yp_v[pl.ds(b0, _L)]
        accs[0] = accs[0] + dp * dp
        accs[1] = accs[1] + dp
        accs[2] = accs[2] + yp
        accs[3] = accs[3] + yp * yp
        for n in range(_NEG):
            e_st = dots[1 + 2 * n] - yst_v[n, pl.ds(b0, _L)]
            accs[4] = accs[4] + e_st * e_st
            e_tt = dots[2 + 2 * n] - ytt_v[n, pl.ds(b0, _L)]
            accs[5] = accs[5] + e_tt * e_tt

    for j in range(6):
        acc_v[j] = accs[j]
    pltpu.sync_copy(acc_v, out_hbm.at[wid])


def kernel(embed_s, embed_t, idx_s, idx_t, probs, adj_mat, adj_t, neg_num):
    del probs, neg_num
    idx_neg = jax.random.randint(
        jax.random.key(42), (_NEG, _B), 0, _N).astype(jnp.int32)
    part = _nsloss_sc(
        idx_s.astype(jnp.int32), idx_t.astype(jnp.int32), idx_neg.reshape(-1),
        embed_s, embed_t, adj_mat.reshape(-1), adj_t.reshape(-1))
    s = jnp.sum(part, axis=(0, 2))
    b = jnp.float32(_B)
    loss_pos = s[0] / b - 2.0 * (s[1] / b) * (s[2] / b) + s[3] / b
    loss_neg = (s[4] + s[5]) / (_NEG * b)
    return (loss_pos + loss_neg) / b
